# baseline (speedup 1.0000x reference)
import functools
import math

import jax
import jax.numpy as jnp
from jax import lax
from jax.experimental import pallas as pl
from jax.experimental.pallas import tpu as pltpu
from jax.experimental.pallas import tpu_sc as plsc

D_MODEL = 1024
LANES = 16
NUM_CORES = 2
NUM_SUBCORES = 16
NUM_WORKERS = NUM_CORES * NUM_SUBCORES
CHUNK = 32


def _emb_body(x_hbm, table_hbm, pe_hbm, out_hbm, idx_v, pe_v, rows_v, sem,
              *, batch, seq):
    scale = math.sqrt(D_MODEL)
    pos_per_w = seq // NUM_WORKERS
    n_chunks = pos_per_w // CHUNK
    wid = lax.axis_index("s") * NUM_CORES + lax.axis_index("c")
    pos0 = wid * pos_per_w

    def chunk_body(c, _):
        p0 = pos0 + c * CHUNK
        pltpu.sync_copy(pe_hbm.at[pl.ds(p0, CHUNK)], pe_v)
        for b in range(batch):
            base = b * seq + p0
            pltpu.sync_copy(x_hbm.at[pl.ds(base, CHUNK)], idx_v)
            pltpu.async_copy(table_hbm.at[idx_v], rows_v, sem).wait()

            def row_body(r, _):
                for j in range(D_MODEL // LANES):
                    sl = pl.ds(j * LANES, LANES)
                    rows_v[r, sl] = rows_v[r, sl] * scale + pe_v[r, sl]
                return 0

            lax.fori_loop(0, CHUNK, row_body, 0)
            pltpu.sync_copy(rows_v, out_hbm.at[pl.ds(base, CHUNK)])
        return 0

    lax.fori_loop(0, n_chunks, chunk_body, 0)


def kernel(x, table, pe):
    batch, seq = x.shape
    x_flat = x.reshape(-1)
    pe2d = pe[0, :seq, :]

    mesh = plsc.VectorSubcoreMesh(core_axis_name="c", subcore_axis_name="s")
    k = pl.kernel(
        functools.partial(_emb_body, batch=batch, seq=seq),
        mesh=mesh,
        out_type=jax.ShapeDtypeStruct((batch * seq, D_MODEL), jnp.float32),
        scratch_types=[
            pltpu.VMEM((CHUNK,), jnp.int32),
            pltpu.VMEM((CHUNK, D_MODEL), jnp.float32),
            pltpu.VMEM((CHUNK, D_MODEL), jnp.float32),
            pltpu.SemaphoreType.DMA,
        ],
    )
    out = k(x_flat, table, pe2d)
    return out.reshape(batch, seq, D_MODEL)


# speedup vs baseline: 1.3866x; 1.3866x over previous
import functools
import math

import jax
import jax.numpy as jnp
from jax import lax
from jax.experimental import pallas as pl
from jax.experimental.pallas import tpu as pltpu
from jax.experimental.pallas import tpu_sc as plsc

D_MODEL = 1024
LANES = 16
NUM_CORES = 2
NUM_SUBCORES = 16
NUM_WORKERS = NUM_CORES * NUM_SUBCORES
CHUNK = 32


def _compute_chunk(gb, pe_v, scale):
    def row_body(r, _):
        for j in range(D_MODEL // LANES):
            sl = pl.ds(j * LANES, LANES)
            gb[r, sl] = gb[r, sl] * scale + pe_v[r, sl]
        return 0

    lax.fori_loop(0, CHUNK, row_body, 0)


def _emb_body(xt_hbm, table_hbm, pe_hbm, out_hbm,
              idx_a, idx_b, pe_v, gb0, gb1, gs0, gs1, ws0, ws1,
              *, batch, seq):
    scale = math.sqrt(D_MODEL)
    pos_per_w = seq // NUM_WORKERS
    n_chunks = pos_per_w // CHUNK
    wid = lax.axis_index("s") * NUM_CORES + lax.axis_index("c")
    g0 = wid * n_chunks

    gbufs = (gb0, gb1)
    gsems = (gs0, gs1)
    wsems = (ws0, ws1)

    def wait_write(q):
        pltpu.make_async_copy(gbufs[q], out_hbm.at[pl.ds(0, CHUNK)],
                              wsems[q]).wait()

    def issue_gather(p, idx_ref, b):
        pltpu.async_copy(table_hbm.at[idx_ref.at[b]], gbufs[p], gsems[p])

    def wait_gather(p, idx_ref, b):
        pltpu.make_async_copy(table_hbm.at[idx_ref.at[b]], gbufs[p],
                              gsems[p]).wait()

    def run_chunk(cc, idx_cur, idx_next, first, last):
        pltpu.sync_copy(pe_hbm.at[pl.ds((g0 + cc) * CHUNK, CHUNK)], pe_v)
        for b in range(batch):
            p = b % 2
            q = 1 - p
            if b == 0:
                @pl.when(jnp.logical_not(first))
                def _():
                    wait_write(q)
                issue_gather(q, idx_cur, b + 1)
            elif b < batch - 1:
                wait_write(q)
                issue_gather(q, idx_cur, b + 1)
            else:
                @pl.when(jnp.logical_not(last))
                def _():
                    wait_write(q)
                    issue_gather(q, idx_next, 0)
            wait_gather(p, idx_cur, b)
            _compute_chunk(gbufs[p], pe_v, scale)
            pltpu.async_copy(
                gbufs[p],
                out_hbm.at[pl.ds(b * seq + (g0 + cc) * CHUNK, CHUNK)],
                wsems[p])

    pltpu.sync_copy(xt_hbm.at[g0], idx_a)
    issue_gather(0, idx_a, 0)

    def outer(i, _):
        base = 2 * i
        pltpu.sync_copy(xt_hbm.at[g0 + base + 1], idx_b)
        run_chunk(base, idx_a, idx_b,
                  first=(base == 0), last=jnp.bool_(False))
        is_last = base + 1 == n_chunks - 1

        @pl.when(jnp.logical_not(is_last))
        def _():
            pltpu.sync_copy(xt_hbm.at[g0 + base + 2], idx_a)
        run_chunk(base + 1, idx_b, idx_a,
                  first=jnp.bool_(False), last=is_last)
        return 0

    lax.fori_loop(0, n_chunks // 2, outer, 0)
    wait_write(0)
    wait_write(1)


def kernel(x, table, pe):
    batch, seq = x.shape
    xt = x.T.reshape(seq // CHUNK, CHUNK, batch).transpose(0, 2, 1)
    pe2d = pe[0, :seq, :]

    mesh = plsc.VectorSubcoreMesh(core_axis_name="c", subcore_axis_name="s")
    k = pl.kernel(
        functools.partial(_emb_body, batch=batch, seq=seq),
        mesh=mesh,
        out_type=jax.ShapeDtypeStruct((batch * seq, D_MODEL), jnp.float32),
        scratch_types=[
            pltpu.VMEM((batch, CHUNK), jnp.int32),
            pltpu.VMEM((batch, CHUNK), jnp.int32),
            pltpu.VMEM((CHUNK, D_MODEL), jnp.float32),
            pltpu.VMEM((CHUNK, D_MODEL), jnp.float32),
            pltpu.VMEM((CHUNK, D_MODEL), jnp.float32),
            pltpu.SemaphoreType.DMA,
            pltpu.SemaphoreType.DMA,
            pltpu.SemaphoreType.DMA,
            pltpu.SemaphoreType.DMA,
        ],
    )
    out = k(xt, table, pe2d)
    return out.reshape(batch, seq, D_MODEL)
